# Initial kernel scaffold; baseline (speedup 1.0000x reference)
import jax
import jax.numpy as jnp
from jax.experimental import pallas as pl


def kernel(x, edge_index, batch, W1, b1, W2, b2):
    raise NotImplementedError("write your pallas kernel here")


# speedup vs baseline: 5.6024x; 5.6024x over previous
import functools

import jax
import jax.numpy as jnp
from jax import lax
from jax.experimental import pallas as pl
from jax.experimental.pallas import tpu as pltpu
from jax.experimental.pallas import tpu_sc as plsc

NN = 10000
EE = 160000
DD = 256
GG = 128

NC = 2
NS = 16

HALF = 5120
NP = NC * HALF
ROWS = HALF // NS
DUMP = HALF

EPS = EE // NS
K = 80
NCHUNK = EPS // K
NV = K // 16

DEGW = 16

R = 320
NBLK = NP // R

_mesh = plsc.VectorSubcoreMesh(core_axis_name="c", subcore_axis_name="s")


def _mask_chunk(dstv, locv, base):
    for t in range(NV):
        d = dstv[pl.ds(16 * t, 16)]
        ok = (d >= base) & (d < base + HALF)
        locv[pl.ds(16 * t, 16)] = jnp.where(ok, d - base, DUMP)


@functools.partial(
    pl.kernel,
    out_type=jax.ShapeDtypeStruct((NP, DEGW), jnp.float32),
    mesh=_mesh,
    scratch_types=[
        pltpu.VMEM_SHARED((HALF + 8, DEGW), jnp.float32),
        pltpu.VMEM((K,), jnp.int32),
        pltpu.VMEM((K,), jnp.int32),
        pltpu.VMEM((K, DEGW), jnp.float32),
    ],
    compiler_params=pltpu.CompilerParams(use_tc_tiling_on_sc=False),
)
def _deg_kernel(dst_hbm, ones_hbm, deg_hbm, acc, dstv, locv, onesv):
    c = lax.axis_index("c")
    s = lax.axis_index("s")
    base = c * HALF

    pltpu.sync_copy(ones_hbm.at[pl.ds(0, K)], onesv)
    for r in range(ROWS // K):
        pltpu.sync_copy(ones_hbm.at[pl.ds(0, K)],
                        acc.at[pl.ds(s * ROWS + r * K, K)])

    @pl.when(s == 0)
    def _():
        pltpu.sync_copy(ones_hbm.at[pl.ds(0, 8)], acc.at[pl.ds(HALF, 8)])

    plsc.subcore_barrier()

    @pl.loop(0, NCHUNK)
    def _(j):
        off = pl.multiple_of(s * EPS + j * K, 8)
        pltpu.sync_copy(dst_hbm.at[pl.ds(off, K)], dstv)
        _mask_chunk(dstv, locv, base)
        pltpu.sync_copy(onesv, acc.at[locv], add=True)

    plsc.subcore_barrier()
    pltpu.sync_copy(acc.at[pl.ds(s * ROWS, ROWS)],
                    deg_hbm.at[pl.ds(base + s * ROWS, ROWS)])


@functools.partial(
    pl.kernel,
    out_type=jax.ShapeDtypeStruct((NP, DD), jnp.float32),
    mesh=_mesh,
    scratch_types=[
        pltpu.VMEM_SHARED((HALF + 8, DD), jnp.float32),
        pltpu.VMEM((K,), jnp.int32),
        pltpu.VMEM((K,), jnp.int32),
        pltpu.VMEM((K,), jnp.int32),
        pltpu.VMEM((K, DD), jnp.float32),
        pltpu.SemaphoreType.DMA,
    ],
    compiler_params=pltpu.CompilerParams(use_tc_tiling_on_sc=False),
)
def _scatter_kernel(src_hbm, dst_hbm, hs_hbm, out_hbm,
                    acc, srcv, dstv, locv, rowsv, sem):
    c = lax.axis_index("c")
    s = lax.axis_index("s")
    base = c * HALF

    pltpu.sync_copy(hs_hbm.at[pl.ds(base + s * ROWS, ROWS)],
                    acc.at[pl.ds(s * ROWS, ROWS)])

    @pl.when(s == 0)
    def _():
        pltpu.sync_copy(hs_hbm.at[pl.ds(0, 8)], acc.at[pl.ds(HALF, 8)])

    plsc.subcore_barrier()

    @pl.loop(0, NCHUNK)
    def _(j):
        off = pl.multiple_of(s * EPS + j * K, 8)
        pltpu.sync_copy(src_hbm.at[pl.ds(off, K)], srcv)
        pltpu.sync_copy(dst_hbm.at[pl.ds(off, K)], dstv)
        _mask_chunk(dstv, locv, base)
        pltpu.async_copy(hs_hbm.at[srcv], rowsv, sem).wait()
        pltpu.sync_copy(rowsv, acc.at[locv], add=True)

    plsc.subcore_barrier()
    pltpu.sync_copy(acc.at[pl.ds(s * ROWS, ROWS)],
                    out_hbm.at[pl.ds(base + s * ROWS, ROWS)])


def _mm1_body(x_ref, w_ref, deg_ref, o_ref):
    dinv = lax.rsqrt(deg_ref[:, 0:1])
    h = jnp.dot(x_ref[:], w_ref[:], preferred_element_type=jnp.float32)
    o_ref[:] = h * dinv


def _mm2_body(a_ref, w_ref, deg_ref, b_ref, o_ref):
    dinv = lax.rsqrt(deg_ref[:, 0:1])
    x2 = dinv * a_ref[:] + b_ref[:]
    h = jnp.dot(x2, w_ref[:], preferred_element_type=jnp.float32)
    o_ref[:] = h * dinv


def _pool_body(a_ref, deg_ref, b_ref, bat_ref, o_ref, sums, cnt):
    i = pl.program_id(0)

    @pl.when(i == 0)
    def _():
        sums[:] = jnp.zeros_like(sums)
        cnt[:] = jnp.zeros_like(cnt)

    dinv = lax.rsqrt(deg_ref[:, 0:1])
    h = dinv * a_ref[:] + b_ref[:]
    gi = lax.broadcasted_iota(jnp.int32, (R, GG), 1)
    P = (bat_ref[:] == gi).astype(jnp.float32)
    dn = (((0,), (0,)), ((), ()))
    sums[:] += lax.dot_general(P, h, dn, preferred_element_type=jnp.float32)
    cnt[:] += lax.dot_general(P, jnp.ones((R, 1), jnp.float32), dn,
                              preferred_element_type=jnp.float32)

    @pl.when(i == NBLK - 1)
    def _():
        o_ref[:] = sums[:] / jnp.maximum(cnt[:], 1.0)


_mm1 = pl.pallas_call(
    _mm1_body,
    grid=(NBLK,),
    in_specs=[
        pl.BlockSpec((R, DD), lambda i: (i, 0)),
        pl.BlockSpec((DD, DD), lambda i: (0, 0)),
        pl.BlockSpec((R, DEGW), lambda i: (i, 0)),
    ],
    out_specs=pl.BlockSpec((R, DD), lambda i: (i, 0)),
    out_shape=jax.ShapeDtypeStruct((NP, DD), jnp.float32),
)

_mm2 = pl.pallas_call(
    _mm2_body,
    grid=(NBLK,),
    in_specs=[
        pl.BlockSpec((R, DD), lambda i: (i, 0)),
        pl.BlockSpec((DD, DD), lambda i: (0, 0)),
        pl.BlockSpec((R, DEGW), lambda i: (i, 0)),
        pl.BlockSpec((1, DD), lambda i: (0, 0)),
    ],
    out_specs=pl.BlockSpec((R, DD), lambda i: (i, 0)),
    out_shape=jax.ShapeDtypeStruct((NP, DD), jnp.float32),
)

_pool = pl.pallas_call(
    _pool_body,
    grid=(NBLK,),
    in_specs=[
        pl.BlockSpec((R, DD), lambda i: (i, 0)),
        pl.BlockSpec((R, DEGW), lambda i: (i, 0)),
        pl.BlockSpec((1, DD), lambda i: (0, 0)),
        pl.BlockSpec((R, 1), lambda i: (i, 0)),
    ],
    out_specs=pl.BlockSpec((GG, DD), lambda i: (0, 0)),
    out_shape=jax.ShapeDtypeStruct((GG, DD), jnp.float32),
    scratch_shapes=[
        pltpu.VMEM((GG, DD), jnp.float32),
        pltpu.VMEM((GG, 1), jnp.float32),
    ],
)


def kernel(x, edge_index, batch, W1, b1, W2, b2):
    src = edge_index[0]
    dst = edge_index[1]
    xp = jnp.zeros((NP, DD), jnp.float32).at[:NN].set(x)
    batp = jnp.full((NP, 1), GG, jnp.int32).at[:NN, 0].set(batch)
    ones_rows = jnp.ones((K, DEGW), jnp.float32)

    deg = _deg_kernel(dst, ones_rows)

    hs1 = _mm1(xp, W1, deg)
    agg1 = _scatter_kernel(src, dst, hs1)

    hs2 = _mm2(agg1, W2, deg, b1.reshape(1, DD))
    agg2 = _scatter_kernel(src, dst, hs2)

    return _pool(agg2, deg, b2.reshape(1, DD), batp)


# speedup vs baseline: 7.9995x; 1.4279x over previous
import functools

import jax
import jax.numpy as jnp
from jax import lax
from jax.experimental import pallas as pl
from jax.experimental.pallas import tpu as pltpu
from jax.experimental.pallas import tpu_sc as plsc

NN = 10000
EE = 160000
DD = 256
GG = 128

NC = 2
NS = 16

HALF = 5120
NP = NC * HALF
ROWS = HALF // NS
DUMP = HALF

EPS = EE // NS
K = 80
NCHUNK = EPS // K
NV = K // 16

DEGW = 16

R = 320
NBLK = NP // R

_mesh = plsc.VectorSubcoreMesh(core_axis_name="c", subcore_axis_name="s")


def _mask_chunk(dstv, locv, base):
    for t in range(NV):
        d = dstv[pl.ds(16 * t, 16)]
        ok = (d >= base) & (d < base + HALF)
        locv[pl.ds(16 * t, 16)] = jnp.where(ok, d - base, DUMP)


@functools.partial(
    pl.kernel,
    out_type=jax.ShapeDtypeStruct((NP, DEGW), jnp.float32),
    mesh=_mesh,
    scratch_types=[
        pltpu.VMEM_SHARED((HALF + 8, DEGW), jnp.float32),
        pltpu.VMEM((K,), jnp.int32),
        pltpu.VMEM((K,), jnp.int32),
        pltpu.VMEM((K, DEGW), jnp.float32),
    ],
    compiler_params=pltpu.CompilerParams(use_tc_tiling_on_sc=False),
)
def _deg_kernel(dst_hbm, ones_hbm, deg_hbm, acc, dstv, locv, onesv):
    c = lax.axis_index("c")
    s = lax.axis_index("s")
    base = c * HALF

    pltpu.sync_copy(ones_hbm.at[pl.ds(0, K)], onesv)
    for r in range(ROWS // K):
        pltpu.sync_copy(ones_hbm.at[pl.ds(0, K)],
                        acc.at[pl.ds(s * ROWS + r * K, K)])

    @pl.when(s == 0)
    def _():
        pltpu.sync_copy(ones_hbm.at[pl.ds(0, 8)], acc.at[pl.ds(HALF, 8)])

    plsc.subcore_barrier()

    @pl.loop(0, NCHUNK)
    def _(j):
        off = pl.multiple_of(s * EPS + j * K, 8)
        pltpu.sync_copy(dst_hbm.at[pl.ds(off, K)], dstv)
        _mask_chunk(dstv, locv, base)
        pltpu.sync_copy(onesv, acc.at[locv], add=True)

    plsc.subcore_barrier()
    pltpu.sync_copy(acc.at[pl.ds(s * ROWS, ROWS)],
                    deg_hbm.at[pl.ds(base + s * ROWS, ROWS)])


@functools.partial(
    pl.kernel,
    out_type=jax.ShapeDtypeStruct((NP, DD), jnp.float32),
    mesh=_mesh,
    scratch_types=[
        pltpu.VMEM_SHARED((HALF + 8, DD), jnp.float32),
        pltpu.VMEM((K,), jnp.int32),
        pltpu.VMEM((K,), jnp.int32),
        pltpu.VMEM((K,), jnp.int32),
        pltpu.VMEM((K,), jnp.int32),
        pltpu.VMEM((K,), jnp.int32),
        pltpu.VMEM((K, DD), jnp.float32),
        pltpu.VMEM((K, DD), jnp.float32),
        pltpu.SemaphoreType.DMA,
        pltpu.SemaphoreType.DMA,
    ],
    compiler_params=pltpu.CompilerParams(use_tc_tiling_on_sc=False),
)
def _scatter_kernel(src_hbm, dst_hbm, hs_hbm, out_hbm,
                    acc, srcv0, srcv1, dstv0, dstv1, locv,
                    rowsv0, rowsv1, sem0, sem1):
    c = lax.axis_index("c")
    s = lax.axis_index("s")
    base = c * HALF

    pltpu.sync_copy(hs_hbm.at[pl.ds(base + s * ROWS, ROWS)],
                    acc.at[pl.ds(s * ROWS, ROWS)])

    @pl.when(s == 0)
    def _():
        pltpu.sync_copy(hs_hbm.at[pl.ds(0, 8)], acc.at[pl.ds(HALF, 8)])

    plsc.subcore_barrier()

    def fetch(t, srcv, dstv, rowsv, sem):
        off = pl.multiple_of(s * EPS + t * K, 8)
        pltpu.sync_copy(src_hbm.at[pl.ds(off, K)], srcv)
        pltpu.sync_copy(dst_hbm.at[pl.ds(off, K)], dstv)
        return pltpu.async_copy(hs_hbm.at[srcv], rowsv, sem)

    def consume(dstv, rowsv, cp):
        cp.wait()
        _mask_chunk(dstv, locv, base)
        pltpu.sync_copy(rowsv, acc.at[locv], add=True)

    fetch(0, srcv0, dstv0, rowsv0, sem0)

    @pl.loop(0, (NCHUNK - 1) // 2)
    def _(j):
        t = j * 2
        cp1 = fetch(t + 1, srcv1, dstv1, rowsv1, sem1)
        consume(dstv0, rowsv0, pltpu.make_async_copy(hs_hbm.at[srcv0],
                                                     rowsv0, sem0))
        cp0 = fetch(t + 2, srcv0, dstv0, rowsv0, sem0)
        consume(dstv1, rowsv1, cp1)

    consume(dstv0, rowsv0, pltpu.make_async_copy(hs_hbm.at[srcv0],
                                                 rowsv0, sem0))

    plsc.subcore_barrier()
    pltpu.sync_copy(acc.at[pl.ds(s * ROWS, ROWS)],
                    out_hbm.at[pl.ds(base + s * ROWS, ROWS)])


def _mm1_body(x_ref, w_ref, deg_ref, o_ref):
    dinv = lax.rsqrt(deg_ref[:, 0:1])
    h = jnp.dot(x_ref[:], w_ref[:], preferred_element_type=jnp.float32)
    o_ref[:] = h * dinv


def _mm2_body(a_ref, w_ref, deg_ref, b_ref, o_ref):
    dinv = lax.rsqrt(deg_ref[:, 0:1])
    x2 = dinv * a_ref[:] + b_ref[:]
    h = jnp.dot(x2, w_ref[:], preferred_element_type=jnp.float32)
    o_ref[:] = h * dinv


def _pool_body(a_ref, deg_ref, b_ref, bat_ref, o_ref, sums, cnt):
    i = pl.program_id(0)

    @pl.when(i == 0)
    def _():
        sums[:] = jnp.zeros_like(sums)
        cnt[:] = jnp.zeros_like(cnt)

    dinv = lax.rsqrt(deg_ref[:, 0:1])
    h = dinv * a_ref[:] + b_ref[:]
    gi = lax.broadcasted_iota(jnp.int32, (R, GG), 1)
    P = (bat_ref[:] == gi).astype(jnp.float32)
    dn = (((0,), (0,)), ((), ()))
    sums[:] += lax.dot_general(P, h, dn, preferred_element_type=jnp.float32)
    cnt[:] += lax.dot_general(P, jnp.ones((R, 1), jnp.float32), dn,
                              preferred_element_type=jnp.float32)

    @pl.when(i == NBLK - 1)
    def _():
        o_ref[:] = sums[:] / jnp.maximum(cnt[:], 1.0)


_mm1 = pl.pallas_call(
    _mm1_body,
    grid=(NBLK,),
    in_specs=[
        pl.BlockSpec((R, DD), lambda i: (i, 0)),
        pl.BlockSpec((DD, DD), lambda i: (0, 0)),
        pl.BlockSpec((R, DEGW), lambda i: (i, 0)),
    ],
    out_specs=pl.BlockSpec((R, DD), lambda i: (i, 0)),
    out_shape=jax.ShapeDtypeStruct((NP, DD), jnp.float32),
)

_mm2 = pl.pallas_call(
    _mm2_body,
    grid=(NBLK,),
    in_specs=[
        pl.BlockSpec((R, DD), lambda i: (i, 0)),
        pl.BlockSpec((DD, DD), lambda i: (0, 0)),
        pl.BlockSpec((R, DEGW), lambda i: (i, 0)),
        pl.BlockSpec((1, DD), lambda i: (0, 0)),
    ],
    out_specs=pl.BlockSpec((R, DD), lambda i: (i, 0)),
    out_shape=jax.ShapeDtypeStruct((NP, DD), jnp.float32),
)

_pool = pl.pallas_call(
    _pool_body,
    grid=(NBLK,),
    in_specs=[
        pl.BlockSpec((R, DD), lambda i: (i, 0)),
        pl.BlockSpec((R, DEGW), lambda i: (i, 0)),
        pl.BlockSpec((1, DD), lambda i: (0, 0)),
        pl.BlockSpec((R, 1), lambda i: (i, 0)),
    ],
    out_specs=pl.BlockSpec((GG, DD), lambda i: (0, 0)),
    out_shape=jax.ShapeDtypeStruct((GG, DD), jnp.float32),
    scratch_shapes=[
        pltpu.VMEM((GG, DD), jnp.float32),
        pltpu.VMEM((GG, 1), jnp.float32),
    ],
)


def kernel(x, edge_index, batch, W1, b1, W2, b2):
    src = edge_index[0]
    dst = edge_index[1]
    xp = jnp.zeros((NP, DD), jnp.float32).at[:NN].set(x)
    batp = jnp.full((NP, 1), GG, jnp.int32).at[:NN, 0].set(batch)
    ones_rows = jnp.ones((K, DEGW), jnp.float32)

    deg = _deg_kernel(dst, ones_rows)

    hs1 = _mm1(xp, W1, deg)
    agg1 = _scatter_kernel(src, dst, hs1)

    hs2 = _mm2(agg1, W2, deg, b1.reshape(1, DD))
    agg2 = _scatter_kernel(src, dst, hs2)

    return _pool(agg2, deg, b2.reshape(1, DD), batp)


# speedup vs baseline: 8.0486x; 1.0061x over previous
import functools

import jax
import jax.numpy as jnp
from jax import lax
from jax.experimental import pallas as pl
from jax.experimental.pallas import tpu as pltpu
from jax.experimental.pallas import tpu_sc as plsc

NN = 10000
EE = 160000
DD = 256
GG = 128

NC = 2
NS = 16

HALF = 5120
NP = NC * HALF
ROWS = HALF // NS
DUMP = HALF

EPS = EE // NS
K = 80
NCHUNK = EPS // K
NV = K // 16

DEGW = 16

R = 320
NBLK = NP // R

_mesh = plsc.VectorSubcoreMesh(core_axis_name="c", subcore_axis_name="s")


def _mask_chunk(dstv, locv, base, nv=NV):
    for t in range(nv):
        d = dstv[pl.ds(16 * t, 16)]
        ok = (d >= base) & (d < base + HALF)
        locv[pl.ds(16 * t, 16)] = jnp.where(ok, d - base, DUMP)


KD = 400
NCHUNKD = EPS // KD
NVD = KD // 16


@functools.partial(
    pl.kernel,
    out_type=jax.ShapeDtypeStruct((NP, DEGW), jnp.float32),
    mesh=_mesh,
    scratch_types=[
        pltpu.VMEM_SHARED((HALF + 8, DEGW), jnp.float32),
        pltpu.VMEM((KD,), jnp.int32),
        pltpu.VMEM((KD,), jnp.int32),
        pltpu.VMEM((KD, DEGW), jnp.float32),
    ],
    compiler_params=pltpu.CompilerParams(use_tc_tiling_on_sc=False),
)
def _deg_kernel(dst_hbm, ones_hbm, deg_hbm, acc, dstv, locv, onesv):
    c = lax.axis_index("c")
    s = lax.axis_index("s")
    base = c * HALF

    pltpu.sync_copy(ones_hbm.at[pl.ds(0, KD)], onesv)
    pltpu.sync_copy(ones_hbm.at[pl.ds(0, ROWS)],
                    acc.at[pl.ds(s * ROWS, ROWS)])

    @pl.when(s == 0)
    def _():
        pltpu.sync_copy(ones_hbm.at[pl.ds(0, 8)], acc.at[pl.ds(HALF, 8)])

    plsc.subcore_barrier()

    @pl.loop(0, NCHUNKD)
    def _(j):
        off = pl.multiple_of(s * EPS + j * KD, 8)
        pltpu.sync_copy(dst_hbm.at[pl.ds(off, KD)], dstv)
        _mask_chunk(dstv, locv, base, NVD)
        pltpu.sync_copy(onesv, acc.at[locv], add=True)

    plsc.subcore_barrier()
    pltpu.sync_copy(acc.at[pl.ds(s * ROWS, ROWS)],
                    deg_hbm.at[pl.ds(base + s * ROWS, ROWS)])


@functools.partial(
    pl.kernel,
    out_type=jax.ShapeDtypeStruct((NP, DD), jnp.float32),
    mesh=_mesh,
    scratch_types=[
        pltpu.VMEM_SHARED((HALF + 8, DD), jnp.float32),
        pltpu.VMEM((K,), jnp.int32),
        pltpu.VMEM((K,), jnp.int32),
        pltpu.VMEM((K,), jnp.int32),
        pltpu.VMEM((K,), jnp.int32),
        pltpu.VMEM((K,), jnp.int32),
        pltpu.VMEM((K, DD), jnp.float32),
        pltpu.VMEM((K, DD), jnp.float32),
        pltpu.SemaphoreType.DMA,
        pltpu.SemaphoreType.DMA,
    ],
    compiler_params=pltpu.CompilerParams(use_tc_tiling_on_sc=False),
)
def _scatter_kernel(src_hbm, dst_hbm, hs_hbm, out_hbm,
                    acc, srcv0, srcv1, dstv0, dstv1, locv,
                    rowsv0, rowsv1, sem0, sem1):
    c = lax.axis_index("c")
    s = lax.axis_index("s")
    base = c * HALF

    pltpu.sync_copy(hs_hbm.at[pl.ds(base + s * ROWS, ROWS)],
                    acc.at[pl.ds(s * ROWS, ROWS)])

    @pl.when(s == 0)
    def _():
        pltpu.sync_copy(hs_hbm.at[pl.ds(0, 8)], acc.at[pl.ds(HALF, 8)])

    plsc.subcore_barrier()

    def fetch(t, srcv, dstv, rowsv, sem):
        off = pl.multiple_of(s * EPS + t * K, 8)
        pltpu.sync_copy(src_hbm.at[pl.ds(off, K)], srcv)
        pltpu.sync_copy(dst_hbm.at[pl.ds(off, K)], dstv)
        return pltpu.async_copy(hs_hbm.at[srcv], rowsv, sem)

    def consume(dstv, rowsv, cp):
        cp.wait()
        _mask_chunk(dstv, locv, base)
        pltpu.sync_copy(rowsv, acc.at[locv], add=True)

    fetch(0, srcv0, dstv0, rowsv0, sem0)

    @pl.loop(0, (NCHUNK - 1) // 2)
    def _(j):
        t = j * 2
        cp1 = fetch(t + 1, srcv1, dstv1, rowsv1, sem1)
        consume(dstv0, rowsv0, pltpu.make_async_copy(hs_hbm.at[srcv0],
                                                     rowsv0, sem0))
        cp0 = fetch(t + 2, srcv0, dstv0, rowsv0, sem0)
        consume(dstv1, rowsv1, cp1)

    consume(dstv0, rowsv0, pltpu.make_async_copy(hs_hbm.at[srcv0],
                                                 rowsv0, sem0))

    plsc.subcore_barrier()
    pltpu.sync_copy(acc.at[pl.ds(s * ROWS, ROWS)],
                    out_hbm.at[pl.ds(base + s * ROWS, ROWS)])


def _mm1_body(x_ref, w_ref, deg_ref, o_ref):
    dinv = lax.rsqrt(deg_ref[:, 0:1])
    h = jnp.dot(x_ref[:], w_ref[:], preferred_element_type=jnp.float32)
    o_ref[:] = h * dinv


def _mm2_body(a_ref, w_ref, deg_ref, b_ref, o_ref):
    dinv = lax.rsqrt(deg_ref[:, 0:1])
    x2 = dinv * a_ref[:] + b_ref[:]
    h = jnp.dot(x2, w_ref[:], preferred_element_type=jnp.float32)
    o_ref[:] = h * dinv


def _pool_body(a_ref, deg_ref, b_ref, bat_ref, o_ref, sums, cnt):
    i = pl.program_id(0)

    @pl.when(i == 0)
    def _():
        sums[:] = jnp.zeros_like(sums)
        cnt[:] = jnp.zeros_like(cnt)

    dinv = lax.rsqrt(deg_ref[:, 0:1])
    h = dinv * a_ref[:] + b_ref[:]
    gi = lax.broadcasted_iota(jnp.int32, (R, GG), 1)
    P = (bat_ref[:] == gi).astype(jnp.float32)
    dn = (((0,), (0,)), ((), ()))
    sums[:] += lax.dot_general(P, h, dn, preferred_element_type=jnp.float32)
    cnt[:] += lax.dot_general(P, jnp.ones((R, 1), jnp.float32), dn,
                              preferred_element_type=jnp.float32)

    @pl.when(i == NBLK - 1)
    def _():
        o_ref[:] = sums[:] / jnp.maximum(cnt[:], 1.0)


_mm1 = pl.pallas_call(
    _mm1_body,
    grid=(NBLK,),
    in_specs=[
        pl.BlockSpec((R, DD), lambda i: (i, 0)),
        pl.BlockSpec((DD, DD), lambda i: (0, 0)),
        pl.BlockSpec((R, DEGW), lambda i: (i, 0)),
    ],
    out_specs=pl.BlockSpec((R, DD), lambda i: (i, 0)),
    out_shape=jax.ShapeDtypeStruct((NP, DD), jnp.float32),
)

_mm2 = pl.pallas_call(
    _mm2_body,
    grid=(NBLK,),
    in_specs=[
        pl.BlockSpec((R, DD), lambda i: (i, 0)),
        pl.BlockSpec((DD, DD), lambda i: (0, 0)),
        pl.BlockSpec((R, DEGW), lambda i: (i, 0)),
        pl.BlockSpec((1, DD), lambda i: (0, 0)),
    ],
    out_specs=pl.BlockSpec((R, DD), lambda i: (i, 0)),
    out_shape=jax.ShapeDtypeStruct((NP, DD), jnp.float32),
)

_pool = pl.pallas_call(
    _pool_body,
    grid=(NBLK,),
    in_specs=[
        pl.BlockSpec((R, DD), lambda i: (i, 0)),
        pl.BlockSpec((R, DEGW), lambda i: (i, 0)),
        pl.BlockSpec((1, DD), lambda i: (0, 0)),
        pl.BlockSpec((R, 1), lambda i: (i, 0)),
    ],
    out_specs=pl.BlockSpec((GG, DD), lambda i: (0, 0)),
    out_shape=jax.ShapeDtypeStruct((GG, DD), jnp.float32),
    scratch_shapes=[
        pltpu.VMEM((GG, DD), jnp.float32),
        pltpu.VMEM((GG, 1), jnp.float32),
    ],
)


def kernel(x, edge_index, batch, W1, b1, W2, b2):
    src = edge_index[0]
    dst = edge_index[1]
    xp = jnp.zeros((NP, DD), jnp.float32).at[:NN].set(x)
    batp = jnp.full((NP, 1), GG, jnp.int32).at[:NN, 0].set(batch)
    ones_rows = jnp.ones((KD, DEGW), jnp.float32)

    deg = _deg_kernel(dst, ones_rows)

    hs1 = _mm1(xp, W1, deg)
    agg1 = _scatter_kernel(src, dst, hs1)

    hs2 = _mm2(agg1, W2, deg, b1.reshape(1, DD))
    agg2 = _scatter_kernel(src, dst, hs2)

    return _pool(agg2, deg, b2.reshape(1, DD), batp)
